# baseline (speedup 1.0000x reference)
import jax
import jax.numpy as jnp
from jax.experimental import pallas as pl

_COLS = 1280
_ROWS = 12288
_BLOCK_ROWS = 1024


def _add_mask_kernel(x_ref, o_ref):
    cols = jax.lax.broadcasted_iota(jnp.int32, x_ref.shape, 1)
    mask = ((cols % 5) < 3).astype(jnp.float32)
    o_ref[...] = x_ref[...] + mask


def kernel(x):
    xf = x.reshape(_ROWS, _COLS)
    out = pl.pallas_call(
        _add_mask_kernel,
        out_shape=jax.ShapeDtypeStruct((_ROWS, _COLS), jnp.float32),
        grid=(_ROWS // _BLOCK_ROWS,),
        in_specs=[pl.BlockSpec((_BLOCK_ROWS, _COLS), lambda i: (i, 0))],
        out_specs=pl.BlockSpec((_BLOCK_ROWS, _COLS), lambda i: (i, 0)),
    )(xf)
    return out.reshape(x.shape)


# speedup vs baseline: 115.3429x; 115.3429x over previous
import jax
import jax.numpy as jnp
from jax.experimental import pallas as pl

_N = 1048576
_BLOCK_N = 65536


def _add_mask_kernel(x_ref, o_ref):
    j = jax.lax.broadcasted_iota(jnp.int32, x_ref.shape, 0)
    o_ref[...] = x_ref[...] + (j < 3).astype(jnp.float32)


def kernel(x):
    xt = jnp.transpose(x, (2, 1, 0))
    out_t = pl.pallas_call(
        _add_mask_kernel,
        out_shape=jax.ShapeDtypeStruct((5, 3, _N), jnp.float32),
        grid=(_N // _BLOCK_N,),
        in_specs=[pl.BlockSpec((5, 3, _BLOCK_N), lambda k: (0, 0, k))],
        out_specs=pl.BlockSpec((5, 3, _BLOCK_N), lambda k: (0, 0, k)),
    )(xt)
    return jnp.transpose(out_t, (2, 1, 0))


# speedup vs baseline: 118.3771x; 1.0263x over previous
import jax
import jax.numpy as jnp
from jax.experimental import pallas as pl

_N = 1048576
_BLOCK_N = 131072


def _add_mask_kernel(x_ref, o_ref):
    j = jax.lax.broadcasted_iota(jnp.int32, x_ref.shape, 0)
    o_ref[...] = x_ref[...] + (j < 3).astype(jnp.float32)


def kernel(x):
    xt = jnp.transpose(x, (2, 1, 0))
    out_t = pl.pallas_call(
        _add_mask_kernel,
        out_shape=jax.ShapeDtypeStruct((5, 3, _N), jnp.float32),
        grid=(_N // _BLOCK_N,),
        in_specs=[pl.BlockSpec((5, 3, _BLOCK_N), lambda k: (0, 0, k))],
        out_specs=pl.BlockSpec((5, 3, _BLOCK_N), lambda k: (0, 0, k)),
    )(xt)
    return jnp.transpose(out_t, (2, 1, 0))


# speedup vs baseline: 118.5319x; 1.0013x over previous
import jax
import jax.numpy as jnp
from jax.experimental import pallas as pl

_N = 1048576
_BLOCK_N = 131072


def _add_mask_kernel(x_ref, o_ref):
    j = jax.lax.broadcasted_iota(jnp.int32, (5, 3, 1), 0)
    o_ref[...] = x_ref[...] + (j < 3).astype(jnp.float32)


def kernel(x):
    xt = jnp.transpose(x, (2, 1, 0))
    out_t = pl.pallas_call(
        _add_mask_kernel,
        out_shape=jax.ShapeDtypeStruct((5, 3, _N), jnp.float32),
        grid=(_N // _BLOCK_N,),
        in_specs=[pl.BlockSpec((5, 3, _BLOCK_N), lambda k: (0, 0, k))],
        out_specs=pl.BlockSpec((5, 3, _BLOCK_N), lambda k: (0, 0, k)),
    )(xt)
    return jnp.transpose(out_t, (2, 1, 0))
